# baseline (speedup 1.0000x reference)
import jax
import jax.numpy as jnp
from jax import lax
from jax.experimental import pallas as pl
from jax.experimental.pallas import tpu as pltpu
from jax.experimental.pallas import tpu_sc as plsc

N = 8192
K = 8
NC = 2
NS = 16
L = 16
CHUNK = N // NS
NV = CHUNK // L
NCAND = NS * L

NEG = float("-inf")
BIG = 0x3FFFFFFF

_DNUMS = lax.GatherDimensionNumbers(
    offset_dims=(), collapsed_slice_dims=(0,), start_index_map=(0,))


def _shuf(v, idx):
    return lax.gather(v, idx[:, None], _DNUMS, (1,),
                      mode=lax.GatherScatterMode.PROMISE_IN_BOUNDS)


def _bfly(v, op, iota):
    for k in (1, 2, 4, 8):
        v = op(v, _shuf(v, iota ^ k))
    return v


def _body(x_hbm, w_hbm, t_hbm,
          x_v, work_v, e_v,
          cv_sh, ci_sh, sum_sh,
          cv_loc, ci_loc, sum_loc,
          row_v, row_i, row_s, row_t):
    c = lax.axis_index("c")
    s = lax.axis_index("s")
    base = s * CHUNK
    src = c * N + base
    iota = lax.iota(jnp.int32, L)
    negv = jnp.full((L,), NEG, jnp.float32)
    lane0 = iota == 0

    pltpu.sync_copy(x_hbm.at[pl.ds(src, CHUNK)], x_v)
    pltpu.sync_copy(x_hbm.at[pl.ds(src, CHUNK)], work_v)

    res_v = jnp.full((L,), NEG, jnp.float32)
    res_i = jnp.full((L,), BIG, jnp.int32)
    lmax = None
    for r in range(K):
        bv = jnp.full((L,), NEG, jnp.float32)
        bi = jnp.full((L,), BIG, jnp.int32)
        for j in range(NV):
            v = work_v[pl.ds(L * j, L)]
            m = v > bv
            bv = jnp.where(m, v, bv)
            bi = jnp.where(m, iota + (L * j), bi)
        gv = _bfly(bv, jnp.maximum, iota)
        gi = _bfly(jnp.where(bv == gv, bi, BIG), jnp.minimum, iota)
        if r == 0:
            lmax = gv
        res_v = jnp.where(iota == r, gv, res_v)
        res_i = jnp.where(iota == r, gi, res_i)
        plsc.store_scatter(work_v, [gi], negv, mask=lane0)

    acc = jnp.zeros((L,), jnp.float32)
    for j in range(NV):
        e = jnp.exp(x_v[pl.ds(L * j, L)] - lmax)
        acc = acc + e
        e_v[pl.ds(L * j, L)] = e
    lsum = _bfly(acc, jnp.add, iota)

    row_v[...] = res_v
    row_i[...] = res_i + base
    row_s[...] = jnp.where(lane0, lsum, 0.0)
    pltpu.sync_copy(row_v, cv_sh.at[pl.ds(s * L, L)])
    pltpu.sync_copy(row_i, ci_sh.at[pl.ds(s * L, L)])
    pltpu.sync_copy(row_s, sum_sh.at[pl.ds(s * L, L)])
    plsc.subcore_barrier()
    pltpu.sync_copy(cv_sh, cv_loc)
    pltpu.sync_copy(ci_sh, ci_loc)
    pltpu.sync_copy(sum_sh, sum_loc)

    mv = jnp.full((L,), NEG, jnp.float32)
    for j in range(NS):
        mv = jnp.maximum(mv, cv_loc[pl.ds(L * j, L)])
    gmax = _bfly(mv, jnp.maximum, iota)

    sacc = jnp.zeros((L,), jnp.float32)
    for j in range(NS):
        mrow = cv_loc[pl.ds(L * j, L)]
        srow = sum_loc[pl.ds(L * j, L)]
        sacc = sacc + jnp.where(lane0, srow * jnp.exp(mrow - gmax), 0.0)
    gsum = _bfly(sacc, jnp.add, iota)
    scale = jnp.exp(lmax - gmax) / gsum

    @pl.when(s == 0)
    def _merge():
        resm = jnp.zeros((L,), jnp.int32)
        for r in range(K):
            bv = jnp.full((L,), NEG, jnp.float32)
            bi = jnp.full((L,), BIG, jnp.int32)
            bp = jnp.full((L,), BIG, jnp.int32)
            for j in range(NS):
                v = cv_loc[pl.ds(L * j, L)]
                gx = ci_loc[pl.ds(L * j, L)]
                m = v > bv
                bv = jnp.where(m, v, bv)
                bi = jnp.where(m, gx, bi)
                bp = jnp.where(m, iota + (L * j), bp)
            gv = _bfly(bv, jnp.maximum, iota)
            vm = bv == gv
            gi = _bfly(jnp.where(vm, bi, BIG), jnp.minimum, iota)
            resm = jnp.where(iota == r, gi, resm)
            gp = _bfly(jnp.where(vm & (bi == gi), bp, BIG), jnp.minimum, iota)
            plsc.store_scatter(cv_loc, [gp], negv, mask=lane0)
        row_t[...] = resm
        pltpu.sync_copy(row_t.at[pl.ds(0, K)], t_hbm.at[pl.ds(c * K, K)])

    for j in range(NV):
        e_v[pl.ds(L * j, L)] = e_v[pl.ds(L * j, L)] * scale
    pltpu.sync_copy(e_v, w_hbm.at[pl.ds(src, CHUNK)])


@jax.jit
def _run(x):
    kern = pl.kernel(
        _body,
        out_type=[
            jax.ShapeDtypeStruct((NC * N,), jnp.float32),
            jax.ShapeDtypeStruct((NC * K,), jnp.int32),
        ],
        mesh=plsc.VectorSubcoreMesh(
            core_axis_name="c", subcore_axis_name="s",
            num_cores=NC, num_subcores=NS),
        scratch_types=[
            pltpu.VMEM((CHUNK,), jnp.float32),
            pltpu.VMEM((CHUNK,), jnp.float32),
            pltpu.VMEM((CHUNK,), jnp.float32),
            pltpu.VMEM_SHARED((NCAND,), jnp.float32),
            pltpu.VMEM_SHARED((NCAND,), jnp.int32),
            pltpu.VMEM_SHARED((NCAND,), jnp.float32),
            pltpu.VMEM((NCAND,), jnp.float32),
            pltpu.VMEM((NCAND,), jnp.int32),
            pltpu.VMEM((NCAND,), jnp.float32),
            pltpu.VMEM((L,), jnp.float32),
            pltpu.VMEM((L,), jnp.int32),
            pltpu.VMEM((L,), jnp.float32),
            pltpu.VMEM((L,), jnp.int32),
        ],
        compiler_params=pltpu.CompilerParams(needs_layout_passes=False),
        name="softmax_top8_sc",
    )
    return kern(x)


def kernel(fc1_logits, fc2_logits):
    w, t = _run(jnp.concatenate([fc1_logits, fc2_logits]))
    return w[:N], w[N:], t[:K], t[K:]


# speedup vs baseline: 1.0724x; 1.0724x over previous
import jax
import jax.numpy as jnp
from jax import lax
from jax.experimental import pallas as pl
from jax.experimental.pallas import tpu as pltpu
from jax.experimental.pallas import tpu_sc as plsc

N = 8192
K = 8
NC = 2
NS = 16
L = 16
CHUNK = N // NS
NV = CHUNK // L
RW = 3 * L

NEG = float("-inf")
BIG = 0x3FFFFFFF

_DNUMS = lax.GatherDimensionNumbers(
    offset_dims=(), collapsed_slice_dims=(0,), start_index_map=(0,))


def _shuf(v, idx):
    return lax.gather(v, idx[:, None], _DNUMS, (1,),
                      mode=lax.GatherScatterMode.PROMISE_IN_BOUNDS)


def _bfly(v, op, iota):
    for k in (1, 2, 4, 8):
        v = op(v, _shuf(v, iota ^ k))
    return v


def _body(x1_hbm, x2_hbm, w_hbm, t_hbm,
          ab_v, e_v, comb_sh, comb_loc, row_c, row_t):
    c = lax.axis_index("c")
    s = lax.axis_index("s")
    base = s * CHUNK
    coff = c * CHUNK
    iota = lax.iota(jnp.int32, L)
    negv = jnp.full((L,), NEG, jnp.float32)
    lane0 = iota == 0

    pltpu.sync_copy(x1_hbm.at[pl.ds(base, CHUNK)], ab_v.at[pl.ds(0, CHUNK)])
    pltpu.sync_copy(x2_hbm.at[pl.ds(base, CHUNK)],
                    ab_v.at[pl.ds(CHUNK, CHUNK)])

    def round_step(r, carry):
        res_v, res_i = carry
        bv = jnp.full((L,), NEG, jnp.float32)
        bi = jnp.full((L,), BIG, jnp.int32)
        for j in range(NV):
            v = ab_v[pl.ds(coff + L * j, L)]
            m = v > bv
            bv = jnp.where(m, v, bv)
            bi = jnp.where(m, iota + (L * j), bi)
        gv = _bfly(bv, jnp.maximum, iota)
        gi = _bfly(jnp.where(bv == gv, bi, BIG), jnp.minimum, iota)
        res_v = jnp.where(iota == r, gv, res_v)
        res_i = jnp.where(iota == r, gi, res_i)
        plsc.store_scatter(ab_v, [gi + coff], negv, mask=lane0)
        return res_v, res_i

    res_v, res_i = lax.fori_loop(
        0, K, round_step,
        (jnp.full((L,), NEG, jnp.float32), jnp.full((L,), BIG, jnp.int32)))

    def restore_step(r, carry):
        rv = jnp.full((L,), 0, jnp.int32) + r
        giv = _shuf(res_i, rv) + coff
        gvv = _shuf(res_v, rv)
        plsc.store_scatter(ab_v, [giv], gvv, mask=lane0)
        return carry
    lax.fori_loop(0, K, restore_step, 0)

    lmax = _shuf(res_v, jnp.zeros((L,), jnp.int32))

    acc = jnp.zeros((L,), jnp.float32)
    for j in range(NV):
        e = jnp.exp(ab_v[pl.ds(coff + L * j, L)] - lmax)
        acc = acc + e
        e_v[pl.ds(L * j, L)] = e
    lsum = _bfly(acc, jnp.add, iota)

    row_c[pl.ds(0, L)] = res_v
    row_c[pl.ds(L, L)] = plsc.bitcast(res_i + base, jnp.float32)
    row_c[pl.ds(2 * L, L)] = jnp.where(lane0, lsum, 0.0)
    pltpu.sync_copy(row_c, comb_sh.at[pl.ds(s * RW, RW)])
    plsc.subcore_barrier()
    pltpu.sync_copy(comb_sh, comb_loc)

    mv = jnp.full((L,), NEG, jnp.float32)
    for j in range(NS):
        mv = jnp.maximum(mv, comb_loc[pl.ds(RW * j, L)])
    gmax = _bfly(mv, jnp.maximum, iota)

    sacc = jnp.zeros((L,), jnp.float32)
    for j in range(NS):
        mrow = comb_loc[pl.ds(RW * j, L)]
        srow = comb_loc[pl.ds(RW * j + 2 * L, L)]
        sacc = sacc + jnp.where(lane0, srow * jnp.exp(mrow - gmax), 0.0)
    gsum = _bfly(sacc, jnp.add, iota)
    scale = jnp.exp(lmax - gmax) / gsum

    @pl.when(s == 0)
    def _merge():
        def merge_step(r, resm):
            bv = jnp.full((L,), NEG, jnp.float32)
            bi = jnp.full((L,), BIG, jnp.int32)
            bp = jnp.full((L,), BIG, jnp.int32)
            for j in range(NS):
                v = comb_loc[pl.ds(RW * j, L)]
                gx = plsc.bitcast(comb_loc[pl.ds(RW * j + L, L)], jnp.int32)
                m = v > bv
                bv = jnp.where(m, v, bv)
                bi = jnp.where(m, gx, bi)
                bp = jnp.where(m, iota + (RW * j), bp)
            gv = _bfly(bv, jnp.maximum, iota)
            vm = bv == gv
            gi = _bfly(jnp.where(vm, bi, BIG), jnp.minimum, iota)
            resm = jnp.where(iota == r, gi, resm)
            gp = _bfly(jnp.where(vm & (bi == gi), bp, BIG), jnp.minimum,
                       iota)
            plsc.store_scatter(comb_loc, [gp], negv, mask=lane0)
            return resm
        resm = lax.fori_loop(0, K, merge_step, jnp.zeros((L,), jnp.int32))
        row_t[...] = resm
        pltpu.sync_copy(row_t.at[pl.ds(0, K)], t_hbm.at[pl.ds(c * K, K)])

    for j in range(NV):
        e_v[pl.ds(L * j, L)] = e_v[pl.ds(L * j, L)] * scale
    pltpu.sync_copy(e_v, w_hbm.at[pl.ds(c * N + base, CHUNK)])


@jax.jit
def _run(x1, x2):
    kern = pl.kernel(
        _body,
        out_type=[
            jax.ShapeDtypeStruct((NC * N,), jnp.float32),
            jax.ShapeDtypeStruct((NC * K,), jnp.int32),
        ],
        mesh=plsc.VectorSubcoreMesh(
            core_axis_name="c", subcore_axis_name="s",
            num_cores=NC, num_subcores=NS),
        scratch_types=[
            pltpu.VMEM((NC * CHUNK,), jnp.float32),
            pltpu.VMEM((CHUNK,), jnp.float32),
            pltpu.VMEM_SHARED((NS * RW,), jnp.float32),
            pltpu.VMEM((NS * RW,), jnp.float32),
            pltpu.VMEM((RW,), jnp.float32),
            pltpu.VMEM((L,), jnp.int32),
        ],
        compiler_params=pltpu.CompilerParams(needs_layout_passes=False),
        name="softmax_top8_sc",
    )
    return kern(x1, x2)


def kernel(fc1_logits, fc2_logits):
    w, t = _run(fc1_logits, fc2_logits)
    return w[:N], w[N:], t[:K], t[K:]


# speedup vs baseline: 1.0737x; 1.0013x over previous
import jax
import jax.numpy as jnp
from jax import lax
from jax.experimental import pallas as pl
from jax.experimental.pallas import tpu as pltpu
from jax.experimental.pallas import tpu_sc as plsc

N = 8192
K = 8
NC = 2
NS = 16
L = 16
CHUNK = N // NS
NV = CHUNK // L
RW = 3 * L

NEG = float("-inf")
BIG = 0x3FFFFFFF

_DNUMS = lax.GatherDimensionNumbers(
    offset_dims=(), collapsed_slice_dims=(0,), start_index_map=(0,))


def _shuf(v, idx):
    return lax.gather(v, idx[:, None], _DNUMS, (1,),
                      mode=lax.GatherScatterMode.PROMISE_IN_BOUNDS)


def _bfly(v, op, iota):
    for k in (1, 2, 4, 8):
        v = op(v, _shuf(v, iota ^ k))
    return v


def _body(x1_hbm, x2_hbm, w_hbm, t_hbm,
          ab_v, e_v, comb_sh, comb_loc, row_c, row_t):
    c = lax.axis_index("c")
    s = lax.axis_index("s")
    base = s * CHUNK
    coff = c * CHUNK
    iota = lax.iota(jnp.int32, L)
    negv = jnp.full((L,), NEG, jnp.float32)
    lane0 = iota == 0

    pltpu.sync_copy(x1_hbm.at[pl.ds(base, CHUNK)], ab_v.at[pl.ds(0, CHUNK)])
    pltpu.sync_copy(x2_hbm.at[pl.ds(base, CHUNK)],
                    ab_v.at[pl.ds(CHUNK, CHUNK)])

    def round_step(r, carry):
        res_v, res_i = carry
        bv = jnp.full((L,), NEG, jnp.float32)
        bi = jnp.full((L,), BIG, jnp.int32)
        for j in range(NV):
            v = ab_v[pl.ds(coff + L * j, L)]
            m = v > bv
            bv = jnp.where(m, v, bv)
            bi = jnp.where(m, iota + (L * j), bi)
        gv = _bfly(bv, jnp.maximum, iota)
        gi = _bfly(jnp.where(bv == gv, bi, BIG), jnp.minimum, iota)
        res_v = jnp.where(iota == r, gv, res_v)
        res_i = jnp.where(iota == r, gi, res_i)
        plsc.store_scatter(ab_v, [gi + coff], negv, mask=lane0)
        return res_v, res_i

    res_v, res_i = lax.fori_loop(
        0, K, round_step,
        (jnp.full((L,), NEG, jnp.float32), jnp.full((L,), BIG, jnp.int32)))

    def restore_step(r, carry):
        rv = jnp.full((L,), 0, jnp.int32) + r
        giv = _shuf(res_i, rv) + coff
        gvv = _shuf(res_v, rv)
        plsc.store_scatter(ab_v, [giv], gvv, mask=lane0)
        return carry
    lax.fori_loop(0, K, restore_step, 0)

    lmax = _shuf(res_v, jnp.zeros((L,), jnp.int32))

    acc = jnp.zeros((L,), jnp.float32)
    for j in range(NV):
        e = jnp.exp(ab_v[pl.ds(coff + L * j, L)] - lmax)
        acc = acc + e
        e_v[pl.ds(L * j, L)] = e
    lsum = _bfly(acc, jnp.add, iota)

    row_c[pl.ds(0, L)] = res_v
    row_c[pl.ds(L, L)] = plsc.bitcast(res_i + base, jnp.float32)
    row_c[pl.ds(2 * L, L)] = jnp.where(lane0, lsum, 0.0)
    pltpu.sync_copy(row_c, comb_sh.at[pl.ds(s * RW, RW)])
    plsc.subcore_barrier()
    pltpu.sync_copy(comb_sh, comb_loc)

    mv = jnp.full((L,), NEG, jnp.float32)
    for j in range(NS):
        mv = jnp.maximum(mv, comb_loc[pl.ds(RW * j, L)])
    gmax = _bfly(mv, jnp.maximum, iota)

    sacc = jnp.zeros((L,), jnp.float32)
    for j in range(NS):
        mrow = comb_loc[pl.ds(RW * j, L)]
        srow = comb_loc[pl.ds(RW * j + 2 * L, L)]
        sacc = sacc + jnp.where(lane0, srow * jnp.exp(mrow - gmax), 0.0)
    gsum = _bfly(sacc, jnp.add, iota)
    scale = jnp.exp(lmax - gmax) / gsum

    @pl.when(s == 0)
    def _merge():
        def merge_step(r, resm):
            bv = jnp.full((L,), NEG, jnp.float32)
            bi = jnp.full((L,), BIG, jnp.int32)
            bp = jnp.full((L,), BIG, jnp.int32)
            for j in range(NS):
                v = comb_loc[pl.ds(RW * j, L)]
                gx = plsc.bitcast(comb_loc[pl.ds(RW * j + L, L)], jnp.int32)
                m = v > bv
                bv = jnp.where(m, v, bv)
                bi = jnp.where(m, gx, bi)
                bp = jnp.where(m, iota + (RW * j), bp)
            gv = _bfly(bv, jnp.maximum, iota)
            vm = bv == gv
            gi = _bfly(jnp.where(vm, bi, BIG), jnp.minimum, iota)
            resm = jnp.where(iota == r, gi, resm)
            gp = _bfly(jnp.where(vm & (bi == gi), bp, BIG), jnp.minimum,
                       iota)
            plsc.store_scatter(comb_loc, [gp], negv, mask=lane0)
            return resm
        resm = lax.fori_loop(0, K, merge_step, jnp.zeros((L,), jnp.int32))
        row_t[...] = resm
        pltpu.sync_copy(row_t.at[pl.ds(0, K)], t_hbm.at[pl.ds(c * K, K)])

    for j in range(NV):
        e_v[pl.ds(L * j, L)] = e_v[pl.ds(L * j, L)] * scale
    pltpu.sync_copy(e_v, w_hbm.at[pl.ds(c * N + base, CHUNK)])


@jax.jit
def _run(x1, x2):
    kern = pl.kernel(
        _body,
        out_type=[
            jax.ShapeDtypeStruct((NC * N,), jnp.float32),
            jax.ShapeDtypeStruct((NC * K,), jnp.int32),
        ],
        mesh=plsc.VectorSubcoreMesh(
            core_axis_name="c", subcore_axis_name="s",
            num_cores=NC, num_subcores=NS),
        scratch_types=[
            pltpu.VMEM((NC * CHUNK,), jnp.float32),
            pltpu.VMEM((CHUNK,), jnp.float32),
            pltpu.VMEM_SHARED((NS * RW,), jnp.float32),
            pltpu.VMEM((NS * RW,), jnp.float32),
            pltpu.VMEM((RW,), jnp.float32),
            pltpu.VMEM((L,), jnp.int32),
        ],
        compiler_params=pltpu.CompilerParams(
            needs_layout_passes=False, skip_device_barrier=True),
        name="softmax_top8_sc",
    )
    return kern(x1, x2)


def kernel(fc1_logits, fc2_logits):
    w, t = _run(fc1_logits, fc2_logits)
    return w[:N], w[N:], t[:K], t[K:]


# speedup vs baseline: 1.0848x; 1.0103x over previous
import jax
import jax.numpy as jnp
from jax import lax
from jax.experimental import pallas as pl
from jax.experimental.pallas import tpu as pltpu
from jax.experimental.pallas import tpu_sc as plsc

N = 8192
K = 8
NC = 2
NS = 16
L = 16
CHUNK = N // NS
NV = CHUNK // L
RW = 3 * L

NEG = float("-inf")
BIG = 0x3FFFFFFF

_DNUMS = lax.GatherDimensionNumbers(
    offset_dims=(), collapsed_slice_dims=(0,), start_index_map=(0,))


def _shuf(v, idx):
    return lax.gather(v, idx[:, None], _DNUMS, (1,),
                      mode=lax.GatherScatterMode.PROMISE_IN_BOUNDS)


def _bfly(v, op, iota):
    for k in (1, 2, 4, 8):
        v = op(v, _shuf(v, iota ^ k))
    return v


def _body(x1_hbm, x2_hbm, w_hbm, t_hbm,
          ab_v, e_v, comb_sh, comb_loc, row_c, row_t):
    c = lax.axis_index("c")
    s = lax.axis_index("s")
    base = s * CHUNK
    coff = c * CHUNK
    iota = lax.iota(jnp.int32, L)
    negv = jnp.full((L,), NEG, jnp.float32)
    lane0 = iota == 0

    pltpu.sync_copy(x1_hbm.at[pl.ds(base, CHUNK)], ab_v.at[pl.ds(0, CHUNK)])
    pltpu.sync_copy(x2_hbm.at[pl.ds(base, CHUNK)],
                    ab_v.at[pl.ds(CHUNK, CHUNK)])

    def round_step(r, carry):
        res_v, res_i = carry
        def sweep(j, carry):
            bv, bi = carry
            v = ab_v[pl.ds(coff + L * j, L)]
            m = v > bv
            return jnp.where(m, v, bv), jnp.where(m, iota + L * j, bi)
        bv, bi = lax.fori_loop(
            0, NV, sweep,
            (jnp.full((L,), NEG, jnp.float32), jnp.full((L,), BIG, jnp.int32)))
        gv = _bfly(bv, jnp.maximum, iota)
        gi = _bfly(jnp.where(bv == gv, bi, BIG), jnp.minimum, iota)
        res_v = jnp.where(iota == r, gv, res_v)
        res_i = jnp.where(iota == r, gi, res_i)
        plsc.store_scatter(ab_v, [gi + coff], negv, mask=lane0)
        return res_v, res_i

    res_v, res_i = lax.fori_loop(
        0, K, round_step,
        (jnp.full((L,), NEG, jnp.float32), jnp.full((L,), BIG, jnp.int32)))

    def restore_step(r, carry):
        rv = jnp.full((L,), 0, jnp.int32) + r
        giv = _shuf(res_i, rv) + coff
        gvv = _shuf(res_v, rv)
        plsc.store_scatter(ab_v, [giv], gvv, mask=lane0)
        return carry
    lax.fori_loop(0, K, restore_step, 0)

    lmax = _shuf(res_v, jnp.zeros((L,), jnp.int32))

    def exp_step(j, acc):
        e = jnp.exp(ab_v[pl.ds(coff + L * j, L)] - lmax)
        e_v[pl.ds(L * j, L)] = e
        return acc + e
    acc = lax.fori_loop(0, NV, exp_step, jnp.zeros((L,), jnp.float32))
    lsum = _bfly(acc, jnp.add, iota)

    row_c[pl.ds(0, L)] = res_v
    row_c[pl.ds(L, L)] = plsc.bitcast(res_i + base, jnp.float32)
    row_c[pl.ds(2 * L, L)] = jnp.where(lane0, lsum, 0.0)
    pltpu.sync_copy(row_c, comb_sh.at[pl.ds(s * RW, RW)])
    plsc.subcore_barrier()
    pltpu.sync_copy(comb_sh, comb_loc)

    def max_step(j, mv):
        return jnp.maximum(mv, comb_loc[pl.ds(RW * j, L)])
    mv = lax.fori_loop(0, NS, max_step, jnp.full((L,), NEG, jnp.float32))
    gmax = _bfly(mv, jnp.maximum, iota)

    def sum_step(j, sacc):
        mrow = comb_loc[pl.ds(RW * j, L)]
        srow = comb_loc[pl.ds(RW * j + 2 * L, L)]
        return sacc + jnp.where(lane0, srow * jnp.exp(mrow - gmax), 0.0)
    sacc = lax.fori_loop(0, NS, sum_step, jnp.zeros((L,), jnp.float32))
    gsum = _bfly(sacc, jnp.add, iota)
    scale = jnp.exp(lmax - gmax) / gsum

    @pl.when(s == 0)
    def _merge():
        def merge_step(r, resm):
            def msweep(j, carry):
                bv, bi, bp = carry
                v = comb_loc[pl.ds(RW * j, L)]
                gx = plsc.bitcast(comb_loc[pl.ds(RW * j + L, L)], jnp.int32)
                m = v > bv
                return (jnp.where(m, v, bv), jnp.where(m, gx, bi),
                        jnp.where(m, iota + RW * j, bp))
            bv, bi, bp = lax.fori_loop(
                0, NS, msweep,
                (jnp.full((L,), NEG, jnp.float32),
                 jnp.full((L,), BIG, jnp.int32),
                 jnp.full((L,), BIG, jnp.int32)))
            gv = _bfly(bv, jnp.maximum, iota)
            vm = bv == gv
            gi = _bfly(jnp.where(vm, bi, BIG), jnp.minimum, iota)
            resm = jnp.where(iota == r, gi, resm)
            gp = _bfly(jnp.where(vm & (bi == gi), bp, BIG), jnp.minimum,
                       iota)
            plsc.store_scatter(comb_loc, [gp], negv, mask=lane0)
            return resm
        resm = lax.fori_loop(0, K, merge_step, jnp.zeros((L,), jnp.int32))
        row_t[...] = resm
        pltpu.sync_copy(row_t.at[pl.ds(0, K)], t_hbm.at[pl.ds(c * K, K)])

    def scale_step(j, carry):
        e_v[pl.ds(L * j, L)] = e_v[pl.ds(L * j, L)] * scale
        return carry
    lax.fori_loop(0, NV, scale_step, 0)
    pltpu.sync_copy(e_v, w_hbm.at[pl.ds(c * N + base, CHUNK)])


@jax.jit
def _run(x1, x2):
    kern = pl.kernel(
        _body,
        out_type=[
            jax.ShapeDtypeStruct((NC * N,), jnp.float32),
            jax.ShapeDtypeStruct((NC * K,), jnp.int32),
        ],
        mesh=plsc.VectorSubcoreMesh(
            core_axis_name="c", subcore_axis_name="s",
            num_cores=NC, num_subcores=NS),
        scratch_types=[
            pltpu.VMEM((NC * CHUNK,), jnp.float32),
            pltpu.VMEM((CHUNK,), jnp.float32),
            pltpu.VMEM_SHARED((NS * RW,), jnp.float32),
            pltpu.VMEM((NS * RW,), jnp.float32),
            pltpu.VMEM((RW,), jnp.float32),
            pltpu.VMEM((L,), jnp.int32),
        ],
        compiler_params=pltpu.CompilerParams(
            needs_layout_passes=False, skip_device_barrier=True),
        name="softmax_top8_sc",
    )
    return kern(x1, x2)


def kernel(fc1_logits, fc2_logits):
    w, t = _run(fc1_logits, fc2_logits)
    return w[:N], w[N:], t[:K], t[K:]


# speedup vs baseline: 1.2215x; 1.1260x over previous
import jax
import jax.numpy as jnp
from jax import lax
from jax.experimental import pallas as pl
from jax.experimental.pallas import tpu as pltpu
from jax.experimental.pallas import tpu_sc as plsc

N = 8192
K = 8
NC = 2
NS = 16
L = 16
CHUNK = N // NS
NV = CHUNK // L
RW = 2 * L

NEG = float("-inf")
BIG = 0x3FFFFFFF

_DNUMS = lax.GatherDimensionNumbers(
    offset_dims=(), collapsed_slice_dims=(0,), start_index_map=(0,))


def _shuf(v, idx):
    return lax.gather(v, idx[:, None], _DNUMS, (1,),
                      mode=lax.GatherScatterMode.PROMISE_IN_BOUNDS)


def _bfly(v, op, iota):
    for k in (1, 2, 4, 8):
        v = op(v, _shuf(v, iota ^ k))
    return v


def _body_sc(x1_hbm, x2_hbm, t_hbm, ab_v, comb_sh, comb_loc, row_c, row_t):
    c = lax.axis_index("c")
    s = lax.axis_index("s")
    base = s * CHUNK
    coff = c * CHUNK
    iota = lax.iota(jnp.int32, L)
    negv = jnp.full((L,), NEG, jnp.float32)
    lane0 = iota == 0

    pltpu.sync_copy(x1_hbm.at[pl.ds(base, CHUNK)], ab_v.at[pl.ds(0, CHUNK)])
    pltpu.sync_copy(x2_hbm.at[pl.ds(base, CHUNK)],
                    ab_v.at[pl.ds(CHUNK, CHUNK)])

    def round_step(r, carry):
        res_v, res_i = carry

        def sweep(j, sc):
            bv, bi = sc
            v = ab_v[pl.ds(coff + L * j, L)]
            m = v > bv
            return jnp.where(m, v, bv), jnp.where(m, iota + L * j, bi)

        bv, bi = lax.fori_loop(
            0, NV, sweep,
            (jnp.full((L,), NEG, jnp.float32),
             jnp.full((L,), BIG, jnp.int32)))
        gv = _bfly(bv, jnp.maximum, iota)
        gi = _bfly(jnp.where(bv == gv, bi, BIG), jnp.minimum, iota)
        res_v = jnp.where(iota == r, gv, res_v)
        res_i = jnp.where(iota == r, gi, res_i)
        plsc.store_scatter(ab_v, [gi + coff], negv, mask=lane0)
        return res_v, res_i

    res_v, res_i = lax.fori_loop(
        0, K, round_step,
        (jnp.full((L,), NEG, jnp.float32), jnp.full((L,), BIG, jnp.int32)))

    row_c[pl.ds(0, L)] = res_v
    row_c[pl.ds(L, L)] = plsc.bitcast(res_i + base, jnp.float32)
    pltpu.sync_copy(row_c, comb_sh.at[pl.ds(s * RW, RW)])
    plsc.subcore_barrier()
    pltpu.sync_copy(comb_sh, comb_loc)

    @pl.when(s == 0)
    def _merge():
        def merge_step(r, resm):
            def msweep(j, sc):
                bv, bi, bp = sc
                v = comb_loc[pl.ds(RW * j, L)]
                gx = plsc.bitcast(comb_loc[pl.ds(RW * j + L, L)], jnp.int32)
                m = v > bv
                return (jnp.where(m, v, bv), jnp.where(m, gx, bi),
                        jnp.where(m, iota + RW * j, bp))

            bv, bi, bp = lax.fori_loop(
                0, NS, msweep,
                (jnp.full((L,), NEG, jnp.float32),
                 jnp.full((L,), BIG, jnp.int32),
                 jnp.full((L,), BIG, jnp.int32)))
            gv = _bfly(bv, jnp.maximum, iota)
            vm = bv == gv
            gi = _bfly(jnp.where(vm, bi, BIG), jnp.minimum, iota)
            resm = jnp.where(iota == r, gi, resm)
            gp = _bfly(jnp.where(vm & (bi == gi), bp, BIG), jnp.minimum,
                       iota)
            plsc.store_scatter(comb_loc, [gp], negv, mask=lane0)
            return resm

        resm = lax.fori_loop(0, K, merge_step, jnp.zeros((L,), jnp.int32))
        row_t[...] = resm
        pltpu.sync_copy(row_t.at[pl.ds(0, K)], t_hbm.at[pl.ds(c * K, K)])


def _body_tc(x1_ref, x2_ref, w1_ref, w2_ref):
    for x_ref, w_ref in ((x1_ref, w1_ref), (x2_ref, w2_ref)):
        x = x_ref[...]
        e = jnp.exp(x - jnp.max(x))
        w_ref[...] = e * (1.0 / jnp.sum(e))


@jax.jit
def _run(x1, x2):
    topk = pl.kernel(
        _body_sc,
        out_type=[jax.ShapeDtypeStruct((NC * K,), jnp.int32)],
        mesh=plsc.VectorSubcoreMesh(
            core_axis_name="c", subcore_axis_name="s",
            num_cores=NC, num_subcores=NS),
        scratch_types=[
            pltpu.VMEM((NC * CHUNK,), jnp.float32),
            pltpu.VMEM_SHARED((NS * RW,), jnp.float32),
            pltpu.VMEM((NS * RW,), jnp.float32),
            pltpu.VMEM((RW,), jnp.float32),
            pltpu.VMEM((L,), jnp.int32),
        ],
        compiler_params=pltpu.CompilerParams(needs_layout_passes=False),
        name="top8_sc",
    )
    t, = topk(x1, x2)
    x1m = x1.reshape(N // 128, 128)
    x2m = x2.reshape(N // 128, 128)
    w1, w2 = pl.pallas_call(
        _body_tc,
        out_shape=[jax.ShapeDtypeStruct((N // 128, 128), jnp.float32),
                   jax.ShapeDtypeStruct((N // 128, 128), jnp.float32)],
        name="softmax_tc",
    )(x1m, x2m)
    return w1.reshape(N), w2.reshape(N), t


def kernel(fc1_logits, fc2_logits):
    w1, w2, t = _run(fc1_logits, fc2_logits)
    return w1, w2, t[:K], t[K:]


# speedup vs baseline: 1.2458x; 1.0199x over previous
import jax
import jax.numpy as jnp
from jax import lax
from jax.experimental import pallas as pl
from jax.experimental.pallas import tpu as pltpu
from jax.experimental.pallas import tpu_sc as plsc

N = 8192
K = 8
NC = 2
NS = 16
L = 16
CHUNK = N // NS
NV = CHUNK // L
RW = 2 * L

NEG = float("-inf")
BIG = 0x3FFFFFFF

_DNUMS = lax.GatherDimensionNumbers(
    offset_dims=(), collapsed_slice_dims=(0,), start_index_map=(0,))


def _shuf(v, idx):
    return lax.gather(v, idx[:, None], _DNUMS, (1,),
                      mode=lax.GatherScatterMode.PROMISE_IN_BOUNDS)


def _bfly(v, op, iota):
    for k in (1, 2, 4, 8):
        v = op(v, _shuf(v, iota ^ k))
    return v


def _body_sc(x1_hbm, x2_hbm, t_hbm, ab_v, comb_sh, comb_loc, row_c, row_t):
    c = lax.axis_index("c")
    s = lax.axis_index("s")
    base = s * CHUNK
    coff = c * CHUNK
    iota = lax.iota(jnp.int32, L)
    negv = jnp.full((L,), NEG, jnp.float32)
    lane0 = iota == 0

    pltpu.sync_copy(x1_hbm.at[pl.ds(base, CHUNK)], ab_v.at[pl.ds(0, CHUNK)])
    pltpu.sync_copy(x2_hbm.at[pl.ds(base, CHUNK)],
                    ab_v.at[pl.ds(CHUNK, CHUNK)])

    def round_step(r, carry):
        res_v, res_i = carry

        def sweep(j, sc):
            bv, bi = sc
            for u in range(4):
                off = L * (4 * j + u)
                v = ab_v[pl.ds(coff + off, L)]
                m = v > bv
                bv = jnp.where(m, v, bv)
                bi = jnp.where(m, iota + off, bi)
            return bv, bi

        bv, bi = lax.fori_loop(
            0, NV // 4, sweep,
            (jnp.full((L,), NEG, jnp.float32),
             jnp.full((L,), BIG, jnp.int32)))
        gv = _bfly(bv, jnp.maximum, iota)
        gi = _bfly(jnp.where(bv == gv, bi, BIG), jnp.minimum, iota)
        res_v = jnp.where(iota == r, gv, res_v)
        res_i = jnp.where(iota == r, gi, res_i)
        plsc.store_scatter(ab_v, [gi + coff], negv, mask=lane0)
        return res_v, res_i

    res_v, res_i = lax.fori_loop(
        0, K, round_step,
        (jnp.full((L,), NEG, jnp.float32), jnp.full((L,), BIG, jnp.int32)))

    row_c[pl.ds(0, L)] = res_v
    row_c[pl.ds(L, L)] = plsc.bitcast(res_i + base, jnp.float32)
    pltpu.sync_copy(row_c, comb_sh.at[pl.ds(s * RW, RW)])
    plsc.subcore_barrier()
    pltpu.sync_copy(comb_sh, comb_loc)

    @pl.when(s == 0)
    def _merge():
        def merge_step(r, resm):
            def msweep(j, sc):
                bv, bi, bp = sc
                v = comb_loc[pl.ds(RW * j, L)]
                gx = plsc.bitcast(comb_loc[pl.ds(RW * j + L, L)], jnp.int32)
                m = v > bv
                return (jnp.where(m, v, bv), jnp.where(m, gx, bi),
                        jnp.where(m, iota + RW * j, bp))

            bv, bi, bp = lax.fori_loop(
                0, NS, msweep,
                (jnp.full((L,), NEG, jnp.float32),
                 jnp.full((L,), BIG, jnp.int32),
                 jnp.full((L,), BIG, jnp.int32)))
            gv = _bfly(bv, jnp.maximum, iota)
            vm = bv == gv
            gi = _bfly(jnp.where(vm, bi, BIG), jnp.minimum, iota)
            resm = jnp.where(iota == r, gi, resm)
            gp = _bfly(jnp.where(vm & (bi == gi), bp, BIG), jnp.minimum,
                       iota)
            plsc.store_scatter(comb_loc, [gp], negv, mask=lane0)
            return resm

        resm = lax.fori_loop(0, K, merge_step, jnp.zeros((L,), jnp.int32))
        row_t[...] = resm
        pltpu.sync_copy(row_t.at[pl.ds(0, K)], t_hbm.at[pl.ds(c * K, K)])


def _body_tc(x1_ref, x2_ref, w1_ref, w2_ref):
    for x_ref, w_ref in ((x1_ref, w1_ref), (x2_ref, w2_ref)):
        x = x_ref[...]
        e = jnp.exp(x - jnp.max(x))
        w_ref[...] = e * (1.0 / jnp.sum(e))


@jax.jit
def _run(x1, x2):
    topk = pl.kernel(
        _body_sc,
        out_type=[jax.ShapeDtypeStruct((NC * K,), jnp.int32)],
        mesh=plsc.VectorSubcoreMesh(
            core_axis_name="c", subcore_axis_name="s",
            num_cores=NC, num_subcores=NS),
        scratch_types=[
            pltpu.VMEM((NC * CHUNK,), jnp.float32),
            pltpu.VMEM_SHARED((NS * RW,), jnp.float32),
            pltpu.VMEM((NS * RW,), jnp.float32),
            pltpu.VMEM((RW,), jnp.float32),
            pltpu.VMEM((L,), jnp.int32),
        ],
        compiler_params=pltpu.CompilerParams(needs_layout_passes=False),
        name="top8_sc",
    )
    w1, w2 = pl.pallas_call(
        _body_tc,
        out_shape=[jax.ShapeDtypeStruct((N,), jnp.float32),
                   jax.ShapeDtypeStruct((N,), jnp.float32)],
        name="softmax_tc",
    )(x1, x2)
    t, = topk(x1, x2)
    return w1, w2, t


def kernel(fc1_logits, fc2_logits):
    w1, w2, t = _run(fc1_logits, fc2_logits)
    return w1, w2, t[:K], t[K:]


# speedup vs baseline: 1.3191x; 1.0588x over previous
import jax
import jax.numpy as jnp
from jax import lax
from jax.experimental import pallas as pl
from jax.experimental.pallas import tpu as pltpu
from jax.experimental.pallas import tpu_sc as plsc

N = 8192
K = 8
NC = 2
NS = 16
L = 16
CHUNK = N // NS
NV = CHUNK // L
RW = 2 * L

NEG = float("-inf")
BIG = 0x3FFFFFFF

_DNUMS = lax.GatherDimensionNumbers(
    offset_dims=(), collapsed_slice_dims=(0,), start_index_map=(0,))


def _shuf(v, idx):
    return lax.gather(v, idx[:, None], _DNUMS, (1,),
                      mode=lax.GatherScatterMode.PROMISE_IN_BOUNDS)


def _bfly(v, op, iota):
    for k in (1, 2, 4, 8):
        v = op(v, _shuf(v, iota ^ k))
    return v


def _body_sc(x1_hbm, x2_hbm, t1_hbm, t2_hbm,
             ab_v, comb_sh, comb_loc, row_c, row_t, anchor_v,
             sem0, sem1, sem2, sem3):
    c = lax.axis_index("c")
    s = lax.axis_index("s")
    base = s * CHUNK
    coff = c * CHUNK
    iota = lax.iota(jnp.int32, L)
    negv = jnp.full((L,), NEG, jnp.float32)
    lane0 = iota == 0

    ca = pltpu.async_copy(t1_hbm, anchor_v.at[pl.ds(0, K)], sem0)
    cb = pltpu.async_copy(t2_hbm, anchor_v.at[pl.ds(K, K)], sem1)
    c1 = pltpu.async_copy(x1_hbm.at[pl.ds(base, CHUNK)],
                          ab_v.at[pl.ds(0, CHUNK)], sem2)
    c2 = pltpu.async_copy(x2_hbm.at[pl.ds(base, CHUNK)],
                          ab_v.at[pl.ds(CHUNK, CHUNK)], sem3)
    ca.wait()
    cb.wait()
    c1.wait()
    c2.wait()

    def round_step(r, carry):
        res_v, res_i = carry

        bv = jnp.full((L,), NEG, jnp.float32)
        bi = jnp.full((L,), BIG, jnp.int32)
        for j in range(NV):
            v = ab_v[pl.ds(coff + L * j, L)]
            m = v > bv
            bv = jnp.where(m, v, bv)
            bi = jnp.where(m, iota + L * j, bi)
        gv = _bfly(bv, jnp.maximum, iota)
        gi = _bfly(jnp.where(bv == gv, bi, BIG), jnp.minimum, iota)
        res_v = jnp.where(iota == r, gv, res_v)
        res_i = jnp.where(iota == r, gi, res_i)
        plsc.store_scatter(ab_v, [gi + coff], negv, mask=lane0)
        return res_v, res_i

    res_v, res_i = lax.fori_loop(
        0, K, round_step,
        (jnp.full((L,), NEG, jnp.float32), jnp.full((L,), BIG, jnp.int32)))

    row_c[pl.ds(0, L)] = res_v
    row_c[pl.ds(L, L)] = plsc.bitcast(res_i + base, jnp.float32)
    pltpu.sync_copy(row_c, comb_sh.at[pl.ds(s * RW, RW)])
    plsc.subcore_barrier()
    pltpu.sync_copy(comb_sh, comb_loc)

    @pl.when(s == 0)
    def _merge():
        def merge_step(r, resm):
            def msweep(j, sc):
                bv, bi, bp = sc
                for u in range(4):
                    jj = 4 * j + u
                    v = comb_loc[pl.ds(RW * jj, L)]
                    gx = plsc.bitcast(comb_loc[pl.ds(RW * jj + L, L)],
                                      jnp.int32)
                    m = v > bv
                    bv = jnp.where(m, v, bv)
                    bi = jnp.where(m, gx, bi)
                    bp = jnp.where(m, iota + RW * jj, bp)
                return bv, bi, bp

            bv, bi, bp = lax.fori_loop(
                0, NS // 4, msweep,
                (jnp.full((L,), NEG, jnp.float32),
                 jnp.full((L,), BIG, jnp.int32),
                 jnp.full((L,), BIG, jnp.int32)))
            gv = _bfly(bv, jnp.maximum, iota)
            vm = bv == gv
            gi = _bfly(jnp.where(vm, bi, BIG), jnp.minimum, iota)
            resm = jnp.where(iota == r, gi, resm)
            gp = _bfly(jnp.where(vm & (bi == gi), bp, BIG), jnp.minimum,
                       iota)
            plsc.store_scatter(comb_loc, [gp], negv, mask=lane0)
            return resm

        resm = lax.fori_loop(0, K, merge_step, jnp.zeros((L,), jnp.int32))
        row_t[...] = resm

        @pl.when(c == 0)
        def _w1():
            pltpu.sync_copy(row_t.at[pl.ds(0, K)], t1_hbm)

        @pl.when(c == 1)
        def _w2():
            pltpu.sync_copy(row_t.at[pl.ds(0, K)], t2_hbm)


def _body_tc(x1_ref, x2_ref, w1_ref, w2_ref):
    for x_ref, w_ref in ((x1_ref, w1_ref), (x2_ref, w2_ref)):
        x = x_ref[...]
        e = jnp.exp(x - jnp.max(x))
        w_ref[...] = e * (1.0 / jnp.sum(e))


@jax.jit
def _run(x1, x2):
    topk = pl.kernel(
        _body_sc,
        out_type=[jax.ShapeDtypeStruct((K,), jnp.int32),
                  jax.ShapeDtypeStruct((K,), jnp.int32)],
        mesh=plsc.VectorSubcoreMesh(
            core_axis_name="c", subcore_axis_name="s",
            num_cores=NC, num_subcores=NS),
        scratch_types=[
            pltpu.VMEM((NC * CHUNK,), jnp.float32),
            pltpu.VMEM_SHARED((NS * RW,), jnp.float32),
            pltpu.VMEM((NS * RW,), jnp.float32),
            pltpu.VMEM((RW,), jnp.float32),
            pltpu.VMEM((L,), jnp.int32),
            pltpu.VMEM((2 * K,), jnp.int32),
            pltpu.SemaphoreType.DMA,
            pltpu.SemaphoreType.DMA,
            pltpu.SemaphoreType.DMA,
            pltpu.SemaphoreType.DMA,
        ],
        compiler_params=pltpu.CompilerParams(needs_layout_passes=False),
        name="top8_sc",
    )
    w1, w2 = pl.pallas_call(
        _body_tc,
        out_shape=[jax.ShapeDtypeStruct((N,), jnp.float32),
                   jax.ShapeDtypeStruct((N,), jnp.float32)],
        name="softmax_tc",
    )(x1, x2)
    t1, t2 = topk(x1, x2)
    return w1, w2, t1, t2


def kernel(fc1_logits, fc2_logits):
    return _run(fc1_logits, fc2_logits)


# speedup vs baseline: 1.3402x; 1.0160x over previous
import jax
import jax.numpy as jnp
from jax import lax
from jax.experimental import pallas as pl
from jax.experimental.pallas import tpu as pltpu
from jax.experimental.pallas import tpu_sc as plsc

N = 8192
K = 8
NC = 2
NS = 16
L = 16
CHUNK = N // NS
NV = CHUNK // L
RW = 2 * L

NEG = float("-inf")
BIG = 0x3FFFFFFF

_DNUMS = lax.GatherDimensionNumbers(
    offset_dims=(), collapsed_slice_dims=(0,), start_index_map=(0,))


def _shuf(v, idx):
    return lax.gather(v, idx[:, None], _DNUMS, (1,),
                      mode=lax.GatherScatterMode.PROMISE_IN_BOUNDS)


def _bfly(v, op, iota):
    for k in (1, 2, 4, 8):
        v = op(v, _shuf(v, iota ^ k))
    return v


def _body_sc(x1_hbm, x2_hbm, t1_hbm, t2_hbm,
             ab_v, comb_sh, comb_loc, row_c, row_t, anchor_v,
             sem0, sem1, sem2, sem3):
    c = lax.axis_index("c")
    s = lax.axis_index("s")
    base = s * CHUNK
    coff = c * CHUNK
    iota = lax.iota(jnp.int32, L)
    negv = jnp.full((L,), NEG, jnp.float32)
    lane0 = iota == 0

    ca = pltpu.async_copy(t1_hbm, anchor_v.at[pl.ds(0, K)], sem0)
    cb = pltpu.async_copy(t2_hbm, anchor_v.at[pl.ds(K, K)], sem1)
    c1 = pltpu.async_copy(x1_hbm.at[pl.ds(base, CHUNK)],
                          ab_v.at[pl.ds(0, CHUNK)], sem2)
    c2 = pltpu.async_copy(x2_hbm.at[pl.ds(base, CHUNK)],
                          ab_v.at[pl.ds(CHUNK, CHUNK)], sem3)
    ca.wait()
    cb.wait()
    c1.wait()
    c2.wait()

    def round_step(r, carry):
        res_v, res_i = carry

        def sweep(j, sc):
            bv, bi = sc
            for u in range(4):
                off = L * (4 * j + u)
                v = ab_v[pl.ds(coff + off, L)]
                m = v > bv
                bv = jnp.where(m, v, bv)
                bi = jnp.where(m, iota + off, bi)
            return bv, bi

        bv, bi = lax.fori_loop(
            0, NV // 4, sweep,
            (jnp.full((L,), NEG, jnp.float32),
             jnp.full((L,), BIG, jnp.int32)))
        gv = _bfly(bv, jnp.maximum, iota)
        gi = _bfly(jnp.where(bv == gv, bi, BIG), jnp.minimum, iota)
        res_v = jnp.where(iota == r, gv, res_v)
        res_i = jnp.where(iota == r, gi, res_i)
        plsc.store_scatter(ab_v, [gi + coff], negv, mask=lane0)
        return res_v, res_i

    res_v, res_i = lax.fori_loop(
        0, K, round_step,
        (jnp.full((L,), NEG, jnp.float32), jnp.full((L,), BIG, jnp.int32)))

    row_c[pl.ds(0, L)] = res_v
    row_c[pl.ds(L, L)] = plsc.bitcast(res_i + base, jnp.float32)
    pltpu.sync_copy(row_c, comb_sh.at[pl.ds(s * RW, RW)])
    plsc.subcore_barrier()
    pltpu.sync_copy(comb_sh, comb_loc)

    @pl.when(s == 0)
    def _merge():
        def merge_step(r, resm):
            def msweep(j, sc):
                bv, bi, bp = sc
                for u in range(4):
                    jj = 4 * j + u
                    v = comb_loc[pl.ds(RW * jj, L)]
                    gx = plsc.bitcast(comb_loc[pl.ds(RW * jj + L, L)],
                                      jnp.int32)
                    m = v > bv
                    bv = jnp.where(m, v, bv)
                    bi = jnp.where(m, gx, bi)
                    bp = jnp.where(m, iota + RW * jj, bp)
                return bv, bi, bp

            bv, bi, bp = lax.fori_loop(
                0, NS // 4, msweep,
                (jnp.full((L,), NEG, jnp.float32),
                 jnp.full((L,), BIG, jnp.int32),
                 jnp.full((L,), BIG, jnp.int32)))
            gv = _bfly(bv, jnp.maximum, iota)
            vm = bv == gv
            gi = _bfly(jnp.where(vm, bi, BIG), jnp.minimum, iota)
            resm = jnp.where(iota == r, gi, resm)
            gp = _bfly(jnp.where(vm & (bi == gi), bp, BIG), jnp.minimum,
                       iota)
            plsc.store_scatter(comb_loc, [gp], negv, mask=lane0)
            return resm

        resm = lax.fori_loop(0, K, merge_step, jnp.zeros((L,), jnp.int32))
        row_t[...] = resm

        @pl.when(c == 0)
        def _w1():
            pltpu.sync_copy(row_t.at[pl.ds(0, K)], t1_hbm)

        @pl.when(c == 1)
        def _w2():
            pltpu.sync_copy(row_t.at[pl.ds(0, K)], t2_hbm)


def _body_tc(x1_ref, x2_ref, w1_ref, w2_ref):
    for x_ref, w_ref in ((x1_ref, w1_ref), (x2_ref, w2_ref)):
        x = x_ref[...]
        e = jnp.exp(x - jnp.max(x))
        w_ref[...] = e * (1.0 / jnp.sum(e))


@jax.jit
def _run(x1, x2):
    topk = pl.kernel(
        _body_sc,
        out_type=[jax.ShapeDtypeStruct((K,), jnp.int32),
                  jax.ShapeDtypeStruct((K,), jnp.int32)],
        mesh=plsc.VectorSubcoreMesh(
            core_axis_name="c", subcore_axis_name="s",
            num_cores=NC, num_subcores=NS),
        scratch_types=[
            pltpu.VMEM((NC * CHUNK,), jnp.float32),
            pltpu.VMEM_SHARED((NS * RW,), jnp.float32),
            pltpu.VMEM((NS * RW,), jnp.float32),
            pltpu.VMEM((RW,), jnp.float32),
            pltpu.VMEM((L,), jnp.int32),
            pltpu.VMEM((2 * K,), jnp.int32),
            pltpu.SemaphoreType.DMA,
            pltpu.SemaphoreType.DMA,
            pltpu.SemaphoreType.DMA,
            pltpu.SemaphoreType.DMA,
        ],
        compiler_params=pltpu.CompilerParams(needs_layout_passes=False),
        name="top8_sc",
    )
    w1, w2 = pl.pallas_call(
        _body_tc,
        out_shape=[jax.ShapeDtypeStruct((N,), jnp.float32),
                   jax.ShapeDtypeStruct((N,), jnp.float32)],
        name="softmax_tc",
    )(x1, x2)
    x1b, x2b, _, _ = lax.optimization_barrier((x1, x2, w1, w2))
    t1, t2 = topk(x1b, x2b)
    return w1, w2, t1, t2


def kernel(fc1_logits, fc2_logits):
    return _run(fc1_logits, fc2_logits)


# speedup vs baseline: 1.3410x; 1.0006x over previous
import jax
import jax.numpy as jnp
from jax import lax
from jax.experimental import pallas as pl
from jax.experimental.pallas import tpu as pltpu
from jax.experimental.pallas import tpu_sc as plsc

N = 8192
K = 8
NC = 2
NS = 16
L = 16
CHUNK = N // NS
NV = CHUNK // L
RW = 2 * L

NEG = float("-inf")
BIG = 0x3FFFFFFF

_DNUMS = lax.GatherDimensionNumbers(
    offset_dims=(), collapsed_slice_dims=(0,), start_index_map=(0,))


def _shuf(v, idx):
    return lax.gather(v, idx[:, None], _DNUMS, (1,),
                      mode=lax.GatherScatterMode.PROMISE_IN_BOUNDS)


def _bfly(v, op, iota):
    for k in (1, 2, 4, 8):
        v = op(v, _shuf(v, iota ^ k))
    return v


def _body_sc(x1_hbm, x2_hbm, t1_hbm, t2_hbm,
             ab_v, comb_sh, comb_loc, row_c, row_t, anchor_v, anchor_f,
             sem0, sem1, sem2, sem3):
    c = lax.axis_index("c")
    s = lax.axis_index("s")
    base = s * CHUNK
    coff = 0
    iota = lax.iota(jnp.int32, L)
    negv = jnp.full((L,), NEG, jnp.float32)
    lane0 = iota == 0

    ca = pltpu.async_copy(t1_hbm, anchor_v.at[pl.ds(0, K)], sem0)
    cb = pltpu.async_copy(t2_hbm, anchor_v.at[pl.ds(K, K)], sem1)
    c1 = pltpu.async_copy(x1_hbm.at[pl.ds(0, K)], anchor_f.at[pl.ds(0, K)],
                          sem2)
    c2 = pltpu.async_copy(x2_hbm.at[pl.ds(0, K)], anchor_f.at[pl.ds(K, K)],
                          sem3)

    @pl.when(c == 0)
    def _in1():
        pltpu.sync_copy(x1_hbm.at[pl.ds(base, CHUNK)],
                        ab_v.at[pl.ds(0, CHUNK)])

    @pl.when(c == 1)
    def _in2():
        pltpu.sync_copy(x2_hbm.at[pl.ds(base, CHUNK)],
                        ab_v.at[pl.ds(0, CHUNK)])

    ca.wait()
    cb.wait()
    c1.wait()
    c2.wait()

    def round_step(r, carry):
        res_v, res_i = carry

        def sweep(j, sc):
            bv, bi = sc
            for u in range(4):
                off = L * (4 * j + u)
                v = ab_v[pl.ds(coff + off, L)]
                m = v > bv
                bv = jnp.where(m, v, bv)
                bi = jnp.where(m, iota + off, bi)
            return bv, bi

        bv, bi = lax.fori_loop(
            0, NV // 4, sweep,
            (jnp.full((L,), NEG, jnp.float32),
             jnp.full((L,), BIG, jnp.int32)))
        gv = _bfly(bv, jnp.maximum, iota)
        gi = _bfly(jnp.where(bv == gv, bi, BIG), jnp.minimum, iota)
        res_v = jnp.where(iota == r, gv, res_v)
        res_i = jnp.where(iota == r, gi, res_i)
        plsc.store_scatter(ab_v, [gi + coff], negv, mask=lane0)
        return res_v, res_i

    res_v, res_i = lax.fori_loop(
        0, K, round_step,
        (jnp.full((L,), NEG, jnp.float32), jnp.full((L,), BIG, jnp.int32)))

    row_c[pl.ds(0, L)] = res_v
    row_c[pl.ds(L, L)] = plsc.bitcast(res_i + base, jnp.float32)
    pltpu.sync_copy(row_c, comb_sh.at[pl.ds(s * RW, RW)])
    plsc.subcore_barrier()
    pltpu.sync_copy(comb_sh, comb_loc)

    @pl.when(s == 0)
    def _merge():
        def merge_step(r, resm):
            def msweep(j, sc):
                bv, bi, bp = sc
                for u in range(4):
                    jj = 4 * j + u
                    v = comb_loc[pl.ds(RW * jj, L)]
                    gx = plsc.bitcast(comb_loc[pl.ds(RW * jj + L, L)],
                                      jnp.int32)
                    m = v > bv
                    bv = jnp.where(m, v, bv)
                    bi = jnp.where(m, gx, bi)
                    bp = jnp.where(m, iota + RW * jj, bp)
                return bv, bi, bp

            bv, bi, bp = lax.fori_loop(
                0, NS // 4, msweep,
                (jnp.full((L,), NEG, jnp.float32),
                 jnp.full((L,), BIG, jnp.int32),
                 jnp.full((L,), BIG, jnp.int32)))
            gv = _bfly(bv, jnp.maximum, iota)
            vm = bv == gv
            gi = _bfly(jnp.where(vm, bi, BIG), jnp.minimum, iota)
            resm = jnp.where(iota == r, gi, resm)
            gp = _bfly(jnp.where(vm & (bi == gi), bp, BIG), jnp.minimum,
                       iota)
            plsc.store_scatter(comb_loc, [gp], negv, mask=lane0)
            return resm

        resm = lax.fori_loop(0, K, merge_step, jnp.zeros((L,), jnp.int32))
        row_t[...] = resm

        @pl.when(c == 0)
        def _w1():
            pltpu.sync_copy(row_t.at[pl.ds(0, K)], t1_hbm)

        @pl.when(c == 1)
        def _w2():
            pltpu.sync_copy(row_t.at[pl.ds(0, K)], t2_hbm)


def _body_tc(x1_ref, x2_ref, w1_ref, w2_ref):
    for x_ref, w_ref in ((x1_ref, w1_ref), (x2_ref, w2_ref)):
        x = x_ref[...]
        e = jnp.exp(x - jnp.max(x))
        w_ref[...] = e * (1.0 / jnp.sum(e))


@jax.jit
def _run(x1, x2):
    topk = pl.kernel(
        _body_sc,
        out_type=[jax.ShapeDtypeStruct((K,), jnp.int32),
                  jax.ShapeDtypeStruct((K,), jnp.int32)],
        mesh=plsc.VectorSubcoreMesh(
            core_axis_name="c", subcore_axis_name="s",
            num_cores=NC, num_subcores=NS),
        scratch_types=[
            pltpu.VMEM((CHUNK,), jnp.float32),
            pltpu.VMEM_SHARED((NS * RW,), jnp.float32),
            pltpu.VMEM((NS * RW,), jnp.float32),
            pltpu.VMEM((RW,), jnp.float32),
            pltpu.VMEM((L,), jnp.int32),
            pltpu.VMEM((2 * K,), jnp.int32),
            pltpu.VMEM((2 * K,), jnp.float32),
            pltpu.SemaphoreType.DMA,
            pltpu.SemaphoreType.DMA,
            pltpu.SemaphoreType.DMA,
            pltpu.SemaphoreType.DMA,
        ],
        compiler_params=pltpu.CompilerParams(needs_layout_passes=False),
        name="top8_sc",
    )
    w1, w2 = pl.pallas_call(
        _body_tc,
        out_shape=[jax.ShapeDtypeStruct((N,), jnp.float32),
                   jax.ShapeDtypeStruct((N,), jnp.float32)],
        name="softmax_tc",
    )(x1, x2)
    x1b, x2b, _, _ = lax.optimization_barrier((x1, x2, w1, w2))
    t1, t2 = topk(x1b, x2b)
    return w1, w2, t1, t2


def kernel(fc1_logits, fc2_logits):
    return _run(fc1_logits, fc2_logits)
